# baseline (speedup 1.0000x reference)
import functools

import jax
import jax.numpy as jnp
import numpy as np
from jax.experimental import pallas as pl
from jax.experimental.pallas import tpu as pltpu


def _hardswish(x):
    return x * jnp.clip(x + 3.0, 0.0, 6.0) * (1.0 / 6.0)


def _kernel(n_pw_chunks, x_ref, bd_ref, dwb_ref, pw_ref, pwb_ref, o_ref,
            xp_ref):
    Nb, H, WC = x_ref.shape
    n_blocks = bd_ref.shape[0]
    CH = pw_ref.shape[-1]
    R = Nb * H
    f32 = jnp.float32

    xp_ref[:, 0:1, :] = jnp.zeros((Nb, 1, WC), f32)
    xp_ref[:, H + 1:H + 2, :] = jnp.zeros((Nb, 1, WC), f32)

    x = x_ref[...].astype(f32).reshape(R, WC)

    for blk in range(n_blocks):
        xp_ref[:, 1:H + 1, :] = x.reshape(Nb, H, WC)

        a = xp_ref[:, 0:H, :].reshape(R, WC)
        c = xp_ref[:, 2:H + 2, :].reshape(R, WC)
        y = (jnp.dot(a, bd_ref[blk, 0], preferred_element_type=f32)
             + jnp.dot(x, bd_ref[blk, 1], preferred_element_type=f32)
             + jnp.dot(c, bd_ref[blk, 2], preferred_element_type=f32))
        y = _hardswish(y + dwb_ref[blk].reshape(1, WC))

        if n_pw_chunks == 1:
            z = jnp.dot(y, pw_ref[blk], preferred_element_type=f32)
        else:
            z = jnp.concatenate(
                [
                    jnp.dot(y[:, k * CH:(k + 1) * CH], pw_ref[blk],
                            preferred_element_type=f32)
                    for k in range(n_pw_chunks)
                ],
                axis=1,
            )
        z = _hardswish(z + pwb_ref[blk].reshape(1, WC))

        x = z + x

    o_ref[...] = x.reshape(Nb, H, WC).astype(o_ref.dtype)


def _band_masks(W, C):
    WC = W * C
    masks = np.zeros((3, WC, WC), np.float32)
    eye_c = np.eye(C, dtype=np.float32)
    for kw in range(3):
        for w in range(W):
            v = w + kw - 1
            if 0 <= v < W:
                masks[kw, v * C:(v + 1) * C, w * C:(w + 1) * C] = eye_c
    return masks


def _bcast_w(v, W):
    n, C = v.shape
    return jnp.broadcast_to(v[:, None, :], (n, W, C)).reshape(n, W * C)


def _build_params(dww, dwb, pww, pwb, W):
    n, _, _, C = dww.shape
    WC = W * C
    dww_f = jnp.broadcast_to(
        dww[:, :, :, None, :], (n, 3, 3, W, C)).reshape(n, 3, 3, WC)
    masks = _band_masks(W, C)
    bd = sum(
        masks[kw][None, None] * dww_f[:, :, kw, None, :]
        for kw in range(3)
    )
    dwb_f = _bcast_w(dwb, W)
    pwb_f = _bcast_w(pwb, W)
    ch = 256 if (WC % 256 == 0 and 256 % C == 0) else WC
    reps = ch // C
    eye_r = np.eye(reps, dtype=np.float32)
    pw_c = (eye_r[None, :, None, :, None]
            * pww[:, None, :, None, :]).reshape(n, ch, ch)
    return bd, dwb_f, pw_c, pwb_f


def kernel(x_nhwc, dww, dwb, pww, pwb):
    N, H, W, C = x_nhwc.shape
    WC = W * C

    bd, dwb_f, pw_c, pwb_f = _build_params(dww, dwb, pww, pwb, W)
    n = bd.shape[0]
    ch = pw_c.shape[-1]
    x_f = x_nhwc.reshape(N, H, WC)

    Nb = next(nb for nb in (16, 8, 4, 2, 1) if N % nb == 0)

    out = pl.pallas_call(
        functools.partial(_kernel, WC // ch),
        out_shape=jax.ShapeDtypeStruct((N, H, WC), x_nhwc.dtype),
        grid_spec=pltpu.PrefetchScalarGridSpec(
            num_scalar_prefetch=0,
            grid=(N // Nb,),
            in_specs=[
                pl.BlockSpec((Nb, H, WC), lambda b: (b, 0, 0)),
                pl.BlockSpec((n, 3, WC, WC), lambda b: (0, 0, 0, 0)),
                pl.BlockSpec((n, WC), lambda b: (0, 0)),
                pl.BlockSpec((n, ch, ch), lambda b: (0, 0, 0)),
                pl.BlockSpec((n, WC), lambda b: (0, 0)),
            ],
            out_specs=pl.BlockSpec((Nb, H, WC), lambda b: (b, 0, 0)),
            scratch_shapes=[pltpu.VMEM((Nb, H + 2, WC), jnp.float32)],
        ),
        compiler_params=pltpu.CompilerParams(
            dimension_semantics=("parallel",),
            vmem_limit_bytes=64 * 1024 * 1024,
        ),
    )(x_f, bd, dwb_f, pw_c, pwb_f)
    return out.reshape(N, H, W, C)


# speedup vs baseline: 1.0086x; 1.0086x over previous
import functools

import jax
import jax.numpy as jnp
import numpy as np
from jax.experimental import pallas as pl
from jax.experimental.pallas import tpu as pltpu


def _hardswish(x):
    return x * jnp.clip(x + 3.0, 0.0, 6.0) * (1.0 / 6.0)


def _kernel(n_pw_chunks, x_ref, bd_ref, dwb_ref, pw_ref, pwb_ref, o_ref,
            xp_ref):
    Nb, H, WC = x_ref.shape
    n_blocks = bd_ref.shape[0]
    CH = pw_ref.shape[-1]
    R = Nb * H
    f32 = jnp.float32

    xp_ref[:, 0:1, :] = jnp.zeros((Nb, 1, WC), f32)
    xp_ref[:, H + 1:H + 2, :] = jnp.zeros((Nb, 1, WC), f32)

    x = x_ref[...].astype(f32).reshape(R, WC)

    for blk in range(n_blocks):
        xp_ref[:, 1:H + 1, :] = x.reshape(Nb, H, WC)

        a = xp_ref[:, 0:H, :].reshape(R, WC)
        c = xp_ref[:, 2:H + 2, :].reshape(R, WC)
        y = (jnp.dot(a, bd_ref[blk, 0], preferred_element_type=f32)
             + jnp.dot(x, bd_ref[blk, 1], preferred_element_type=f32)
             + jnp.dot(c, bd_ref[blk, 2], preferred_element_type=f32))
        y = _hardswish(y + dwb_ref[blk].reshape(1, WC))

        if n_pw_chunks == 1:
            z = jnp.dot(y, pw_ref[blk], preferred_element_type=f32)
        else:
            z = jnp.concatenate(
                [
                    jnp.dot(y[:, k * CH:(k + 1) * CH], pw_ref[blk],
                            preferred_element_type=f32)
                    for k in range(n_pw_chunks)
                ],
                axis=1,
            )
        z = _hardswish(z + pwb_ref[blk].reshape(1, WC))

        x = z + x

    o_ref[...] = x.reshape(Nb, H, WC).astype(o_ref.dtype)


def _band_masks(W, C):
    WC = W * C
    masks = np.zeros((3, WC, WC), np.float32)
    eye_c = np.eye(C, dtype=np.float32)
    for kw in range(3):
        for w in range(W):
            v = w + kw - 1
            if 0 <= v < W:
                masks[kw, v * C:(v + 1) * C, w * C:(w + 1) * C] = eye_c
    return masks


def _pw_mask(reps, C):
    m = np.zeros((reps * C, reps * C), np.float32)
    for u in range(reps):
        m[u * C:(u + 1) * C, u * C:(u + 1) * C] = 1.0
    return m


def _build_params(dww, dwb, pww, pwb, W):
    n, _, _, C = dww.shape
    WC = W * C
    dww_f = jnp.tile(dww[:, :, :, None, :], (1, 1, 1, W, 1)).reshape(n, 3, 3, WC)
    masks = _band_masks(W, C)
    bd = sum(
        masks[kw][None, None] * dww_f[:, :, kw, None, :]
        for kw in range(3)
    )
    dwb_f = jnp.tile(dwb, (1, W))
    pwb_f = jnp.tile(pwb, (1, W))
    ch = 256 if (WC % 256 == 0 and 256 % C == 0) else WC
    reps = ch // C
    pw_c = _pw_mask(reps, C) * jnp.tile(pww, (1, reps, reps))
    return bd, dwb_f, pw_c, pwb_f


def kernel(x_nhwc, dww, dwb, pww, pwb):
    N, H, W, C = x_nhwc.shape
    WC = W * C

    bd, dwb_f, pw_c, pwb_f = _build_params(dww, dwb, pww, pwb, W)
    n = bd.shape[0]
    ch = pw_c.shape[-1]
    x_f = x_nhwc.reshape(N, H, WC)

    Nb = next(nb for nb in (16, 8, 4, 2, 1) if N % nb == 0)

    out = pl.pallas_call(
        functools.partial(_kernel, WC // ch),
        out_shape=jax.ShapeDtypeStruct((N, H, WC), x_nhwc.dtype),
        grid_spec=pltpu.PrefetchScalarGridSpec(
            num_scalar_prefetch=0,
            grid=(N // Nb,),
            in_specs=[
                pl.BlockSpec((Nb, H, WC), lambda b: (b, 0, 0)),
                pl.BlockSpec((n, 3, WC, WC), lambda b: (0, 0, 0, 0)),
                pl.BlockSpec((n, WC), lambda b: (0, 0)),
                pl.BlockSpec((n, ch, ch), lambda b: (0, 0, 0)),
                pl.BlockSpec((n, WC), lambda b: (0, 0)),
            ],
            out_specs=pl.BlockSpec((Nb, H, WC), lambda b: (b, 0, 0)),
            scratch_shapes=[pltpu.VMEM((Nb, H + 2, WC), jnp.float32)],
        ),
        compiler_params=pltpu.CompilerParams(
            dimension_semantics=("parallel",),
            vmem_limit_bytes=64 * 1024 * 1024,
        ),
    )(x_f, bd, dwb_f, pw_c, pwb_f)
    return out.reshape(N, H, W, C)
